# baseline (speedup 1.0000x reference)
import functools

import jax
import jax.numpy as jnp
from jax import lax
from jax.experimental import pallas as pl
from jax.experimental.pallas import tpu as pltpu
from jax.experimental.pallas import tpu_sc as plsc

_BATCH = 16384
_RANK = 64
_NENT = 1000000
_LANES = 16
_BLK = 32

_info = plsc.get_sparse_core_info()
_NC, _NS = _info.num_cores, _info.num_subcores
_NW = _NC * _NS
_CHUNK = _BATCH // _NW
_NBLK = _CHUNK // _BLK


def _distmult_body(s_idx_hbm, p_idx_hbm, o_idx_hbm, so2_hbm, emb_p_hbm,
                   out_hbm, s_idx_v, p_idx_v, o_idx_v, s_rows, p_rows,
                   o_rows, part_v, out_v, sem0, sem1):
    wid = lax.axis_index("s") * _NC + lax.axis_index("c")
    base = wid * _CHUNK

    pltpu.sync_copy(s_idx_hbm.at[pl.ds(base, _CHUNK)], s_idx_v)
    pltpu.sync_copy(p_idx_hbm.at[pl.ds(base, _CHUNK)], p_idx_v)
    pltpu.sync_copy(o_idx_hbm.at[pl.ds(base, _CHUNK)], o_idx_v)

    sems = [sem0, sem1]
    lane = lax.iota(jnp.int32, _LANES)
    half = jnp.int32(_NENT // 2)

    def fire(g, slot, sem):
        b0 = g * _BLK
        for v0 in range(0, _BLK, _LANES):
            ev_s = s_idx_v[pl.ds(b0 + v0, _LANES)]
            ev_p = p_idx_v[pl.ds(b0 + v0, _LANES)]
            ev_o = o_idx_v[pl.ds(b0 + v0, _LANES)]
            hi_s = (ev_s >= half).astype(jnp.int32)
            hi_o = (ev_o >= half).astype(jnp.int32)
            ev_s2 = ev_s - hi_s * half
            ev_o2 = ev_o - hi_o * half
            for j in range(_LANES):
                pltpu.make_async_copy(
                    so2_hbm.at[hi_s[j], pl.ds(ev_s2[j], 1)],
                    s_rows.at[slot, pl.ds(v0 + j, 1)], sem).start()
                pltpu.make_async_copy(
                    emb_p_hbm.at[pl.ds(ev_p[j], 1)],
                    p_rows.at[slot, pl.ds(v0 + j, 1)], sem).start()
                pltpu.make_async_copy(
                    so2_hbm.at[hi_o[j], pl.ds(ev_o2[j], 1)],
                    o_rows.at[slot, pl.ds(v0 + j, 1)], sem).start()

    def drain(slot, sem):
        pltpu.make_async_copy(
            so2_hbm.at[0, pl.ds(0, _BLK)], s_rows.at[slot], sem).wait()
        pltpu.make_async_copy(
            so2_hbm.at[0, pl.ds(0, _BLK)], p_rows.at[slot], sem).wait()
        pltpu.make_async_copy(
            so2_hbm.at[0, pl.ds(0, _BLK)], o_rows.at[slot], sem).wait()

    def compute(g, slot):
        b0 = g * _BLK
        for v0 in range(0, _BLK, _LANES):
            for j in range(_LANES):
                b = v0 + j
                acc = (s_rows[slot, b, pl.ds(0, _LANES)]
                       * p_rows[slot, b, pl.ds(0, _LANES)]
                       * o_rows[slot, b, pl.ds(0, _LANES)])
                for k in range(1, _RANK // _LANES):
                    acc = acc + (s_rows[slot, b, pl.ds(k * _LANES, _LANES)]
                                 * p_rows[slot, b, pl.ds(k * _LANES, _LANES)]
                                 * o_rows[slot, b, pl.ds(k * _LANES,
                                                         _LANES)])
                part_v[j, pl.ds(0, _LANES)] = acc
            out_vec = plsc.load_gather(part_v, [lane, jnp.full((_LANES,), 0,
                                                               jnp.int32)])
            for i in range(1, _LANES):
                out_vec = out_vec + plsc.load_gather(
                    part_v, [lane, jnp.full((_LANES,), i, jnp.int32)])
            out_v[pl.ds(b0 + v0, _LANES)] = out_vec

    def body(g, _):
        slot = lax.rem(g, 2)

        @pl.when(g < _NBLK)
        def _fire():
            @pl.when(slot == 0)
            def _():
                fire(g, 0, sems[0])
            @pl.when(slot == 1)
            def _():
                fire(g, 1, sems[1])

        @pl.when(g > 0)
        def _consume():
            pslot = lax.rem(g + 1, 2)

            @pl.when(pslot == 0)
            def _():
                drain(0, sems[0])
                compute(g - 1, 0)
            @pl.when(pslot == 1)
            def _():
                drain(1, sems[1])
                compute(g - 1, 1)
        return _

    lax.fori_loop(0, _NBLK + 1, body, None)

    pltpu.sync_copy(out_v, out_hbm.at[pl.ds(base, _CHUNK)])


@jax.jit
def kernel(s_idx, p_idx, o_idx, emb_so, emb_p):
    mesh = plsc.VectorSubcoreMesh(core_axis_name="c", subcore_axis_name="s")
    run = pl.kernel(
        _distmult_body,
        out_type=jax.ShapeDtypeStruct((_BATCH,), jnp.float32),
        mesh=mesh,
        compiler_params=pltpu.CompilerParams(needs_layout_passes=False,
                                             use_tc_tiling_on_sc=True),
        scratch_types=[
            pltpu.VMEM((_CHUNK,), jnp.int32),
            pltpu.VMEM((_CHUNK,), jnp.int32),
            pltpu.VMEM((_CHUNK,), jnp.int32),
            pltpu.VMEM((2, _BLK, _RANK), jnp.float32),
            pltpu.VMEM((2, _BLK, _RANK), jnp.float32),
            pltpu.VMEM((2, _BLK, _RANK), jnp.float32),
            pltpu.VMEM((_LANES, _LANES), jnp.float32),
            pltpu.VMEM((_CHUNK,), jnp.float32),
            pltpu.SemaphoreType.DMA,
            pltpu.SemaphoreType.DMA,
        ],
    )
    so2 = emb_so.reshape(2, _NENT // 2, _RANK)
    return run(s_idx.astype(jnp.int32), p_idx.astype(jnp.int32),
               o_idx.astype(jnp.int32), so2, emb_p)


# speedup vs baseline: 1.0345x; 1.0345x over previous
import functools

import jax
import jax.numpy as jnp
from jax import lax
from jax.experimental import pallas as pl
from jax.experimental.pallas import tpu as pltpu
from jax.experimental.pallas import tpu_sc as plsc

_BATCH = 16384
_RANK = 64
_NENT = 1000000
_LANES = 16
_BLK = 16

_info = plsc.get_sparse_core_info()
_NC, _NS = _info.num_cores, _info.num_subcores
_NW = _NC * _NS
_CHUNK = _BATCH // _NW
_NBLK = _CHUNK // _BLK


def _distmult_body(s_idx_hbm, p_idx_hbm, o_idx_hbm, so2_hbm, emb_p_hbm,
                   out_hbm, s_idx_v, p_idx_v, o_idx_v, s_rows, p_rows,
                   o_rows, part_v, out_v, sem0, sem1):
    wid = lax.axis_index("s") * _NC + lax.axis_index("c")
    base = wid * _CHUNK

    pltpu.sync_copy(s_idx_hbm.at[pl.ds(base, _CHUNK)], s_idx_v)
    pltpu.sync_copy(p_idx_hbm.at[pl.ds(base, _CHUNK)], p_idx_v)
    pltpu.sync_copy(o_idx_hbm.at[pl.ds(base, _CHUNK)], o_idx_v)

    sems = [sem0, sem1]
    lane = lax.iota(jnp.int32, _LANES)
    half = jnp.int32(_NENT // 2)

    def fire(g, slot, sem):
        b0 = g * _BLK
        for v0 in range(0, _BLK, _LANES):
            ev_s = s_idx_v[pl.ds(b0 + v0, _LANES)]
            ev_p = p_idx_v[pl.ds(b0 + v0, _LANES)]
            ev_o = o_idx_v[pl.ds(b0 + v0, _LANES)]
            hi_s = (ev_s >= half).astype(jnp.int32)
            hi_o = (ev_o >= half).astype(jnp.int32)
            ev_s2 = ev_s - hi_s * half
            ev_o2 = ev_o - hi_o * half
            for j in range(_LANES):
                pltpu.make_async_copy(
                    so2_hbm.at[hi_s[j], pl.ds(ev_s2[j], 1)],
                    s_rows.at[slot, pl.ds(v0 + j, 1)], sem).start()
                pltpu.make_async_copy(
                    emb_p_hbm.at[pl.ds(ev_p[j], 1)],
                    p_rows.at[slot, pl.ds(v0 + j, 1)], sem).start()
                pltpu.make_async_copy(
                    so2_hbm.at[hi_o[j], pl.ds(ev_o2[j], 1)],
                    o_rows.at[slot, pl.ds(v0 + j, 1)], sem).start()

    def drain(slot, sem):
        pltpu.make_async_copy(
            so2_hbm.at[0, pl.ds(0, _BLK)], s_rows.at[slot], sem).wait()
        pltpu.make_async_copy(
            so2_hbm.at[0, pl.ds(0, _BLK)], p_rows.at[slot], sem).wait()
        pltpu.make_async_copy(
            so2_hbm.at[0, pl.ds(0, _BLK)], o_rows.at[slot], sem).wait()

    def compute(g, slot):
        b0 = g * _BLK
        for v0 in range(0, _BLK, _LANES):
            for j in range(_LANES):
                b = v0 + j
                acc = (s_rows[slot, b, pl.ds(0, _LANES)]
                       * p_rows[slot, b, pl.ds(0, _LANES)]
                       * o_rows[slot, b, pl.ds(0, _LANES)])
                for k in range(1, _RANK // _LANES):
                    acc = acc + (s_rows[slot, b, pl.ds(k * _LANES, _LANES)]
                                 * p_rows[slot, b, pl.ds(k * _LANES, _LANES)]
                                 * o_rows[slot, b, pl.ds(k * _LANES,
                                                         _LANES)])
                part_v[j, pl.ds(0, _LANES)] = acc
            out_vec = plsc.load_gather(part_v, [lane, jnp.full((_LANES,), 0,
                                                               jnp.int32)])
            for i in range(1, _LANES):
                out_vec = out_vec + plsc.load_gather(
                    part_v, [lane, jnp.full((_LANES,), i, jnp.int32)])
            out_v[pl.ds(b0 + v0, _LANES)] = out_vec

    def body(g, _):
        slot = lax.rem(g, 2)

        @pl.when(g < _NBLK)
        def _fire():
            @pl.when(slot == 0)
            def _():
                fire(g, 0, sems[0])
            @pl.when(slot == 1)
            def _():
                fire(g, 1, sems[1])

        @pl.when(g > 0)
        def _consume():
            pslot = lax.rem(g + 1, 2)

            @pl.when(pslot == 0)
            def _():
                drain(0, sems[0])
                compute(g - 1, 0)
            @pl.when(pslot == 1)
            def _():
                drain(1, sems[1])
                compute(g - 1, 1)
        return _

    lax.fori_loop(0, _NBLK + 1, body, None)

    pltpu.sync_copy(out_v, out_hbm.at[pl.ds(base, _CHUNK)])


@jax.jit
def kernel(s_idx, p_idx, o_idx, emb_so, emb_p):
    mesh = plsc.VectorSubcoreMesh(core_axis_name="c", subcore_axis_name="s")
    run = pl.kernel(
        _distmult_body,
        out_type=jax.ShapeDtypeStruct((_BATCH,), jnp.float32),
        mesh=mesh,
        compiler_params=pltpu.CompilerParams(needs_layout_passes=False,
                                             use_tc_tiling_on_sc=True),
        scratch_types=[
            pltpu.VMEM((_CHUNK,), jnp.int32),
            pltpu.VMEM((_CHUNK,), jnp.int32),
            pltpu.VMEM((_CHUNK,), jnp.int32),
            pltpu.VMEM((2, _BLK, _RANK), jnp.float32),
            pltpu.VMEM((2, _BLK, _RANK), jnp.float32),
            pltpu.VMEM((2, _BLK, _RANK), jnp.float32),
            pltpu.VMEM((_LANES, _LANES), jnp.float32),
            pltpu.VMEM((_CHUNK,), jnp.float32),
            pltpu.SemaphoreType.DMA,
            pltpu.SemaphoreType.DMA,
        ],
    )
    so2 = emb_so.reshape(2, _NENT // 2, _RANK)
    return run(s_idx.astype(jnp.int32), p_idx.astype(jnp.int32),
               o_idx.astype(jnp.int32), so2, emb_p)
